# baseline (speedup 1.0000x reference)
import functools

import jax
import jax.numpy as jnp
from jax.experimental import pallas as pl
from jax.experimental.pallas import tpu as pltpu

_K = 3


def _sa_body(x_ref, w1_ref, wc_ref, b_ref, o_ref, *, H, W):
    NB, C, L = x_ref.shape
    xb = x_ref[...]
    wv = w1_ref[...]

    z1 = jnp.sum(xb * wv[None, :, :], axis=1) + b_ref[0]
    avg = jnp.sum(xb, axis=1) * (1.0 / C)
    mx = jnp.max(xb, axis=1)

    pos = jax.lax.broadcasted_iota(jnp.int32, (NB, L), 1)
    col = pos % W
    row = pos // W
    cg = col >= 1
    cl = col < (W - 1)
    rg = row >= 1
    rl = row < (H - 1)
    masks = {
        (-1, -1): rg & cg, (-1, 0): rg, (-1, 1): rg & cl,
        (0, -1): cg, (0, 0): None, (0, 1): cl,
        (1, -1): rl & cg, (1, 0): rl, (1, 1): rl & cl,
    }

    acc = None
    for kh in range(_K):
        dh = kh - 1
        for kw in range(_K):
            dw = kw - 1
            wa = wc_ref[kh * _K + kw]
            wb = wc_ref[9 + kh * _K + kw]
            wm = wc_ref[18 + kh * _K + kw]
            t = z1 * wa + avg * wb + mx * wm
            off = dh * W + dw
            if off != 0:
                t = pltpu.roll(t, (-off) % L, axis=1)
            m = masks[(dh, dw)]
            if m is not None:
                t = jnp.where(m, t, 0.0)
            acc = t if acc is None else acc + t

    o_ref[...] = jax.nn.sigmoid(acc + b_ref[1])


def kernel(x, w1, b1, g1, be1, m1, v1, wc, b2, g2, be2, m2, v2):
    N, C, H, W = x.shape
    L = H * W
    eps = 1e-5

    s1 = (g1 / jnp.sqrt(v1 + eps)).reshape(())
    s2 = (g2 / jnp.sqrt(v2 + eps)).reshape(())
    w1v = (w1.reshape(C, 1) * s1).astype(jnp.float32)
    wcf = (wc.reshape(_K * _K * 3) * s2).astype(jnp.float32)
    t1 = be1.reshape(()) - m1.reshape(()) * s1
    t2 = be2.reshape(()) - m2.reshape(()) * s2
    bias = jnp.stack(
        [b1.reshape(()) * s1 + t1, b2.reshape(()) * s2 + t2]
    ).astype(jnp.float32)

    xf = x.reshape(N, C, L).astype(jnp.float32)

    nb = N if N <= 16 else 16
    grid = (pl.cdiv(N, nb),)

    out = pl.pallas_call(
        functools.partial(_sa_body, H=H, W=W),
        out_shape=jax.ShapeDtypeStruct((N, L), jnp.float32),
        grid=grid,
        in_specs=[
            pl.BlockSpec((nb, C, L), lambda n: (n, 0, 0)),
            pl.BlockSpec((C, 1), lambda n: (0, 0)),
            pl.BlockSpec(memory_space=pltpu.MemorySpace.SMEM),
            pl.BlockSpec(memory_space=pltpu.MemorySpace.SMEM),
        ],
        out_specs=pl.BlockSpec((nb, L), lambda n: (n, 0)),
        compiler_params=pltpu.CompilerParams(
            dimension_semantics=("parallel",)),
    )(xf, w1v, wcf, bias)
    return out.reshape(N, 1, H, W)
